# Initial kernel scaffold; baseline (speedup 1.0000x reference)
import functools
import math

import numpy as np
import jax
import jax.numpy as jnp
from jax import lax
from jax.experimental import pallas as pl
from jax.experimental.pallas import tpu as pltpu
from jax.experimental.pallas import tpu_sc as plsc

F32 = jnp.float32
NE = 8
FCH = 18
BATCH = 4

_PCE = {
    0: dict(C=32, ps=16, hp=14, N=784, Npad=1024, Ccap=123, STRIDE=128),
    1: dict(C=32, ps=16, hp=14, N=784, Npad=1024, Ccap=123, STRIDE=128),
    3: dict(C=64, ps=4, hp=28, N=3136, Npad=3328, Ccap=490, STRIDE=512),
    5: dict(C=128, ps=2, hp=28, N=3136, Npad=3328, Ccap=490, STRIDE=512),
    7: dict(C=256, ps=1, hp=28, N=3136, Npad=3328, Ccap=490, STRIDE=512),
}


def _silu(x):
    return x * jax.nn.sigmoid(x)


def _dot(a, b):
    return lax.dot_general(a, b, (((a.ndim - 1,), (0,)), ((), ())),
                           preferred_element_type=F32)


def _posfeat_np(hp):
    ys = (np.arange(hp) + 0.5) / hp
    yy, xx = np.meshgrid(ys, ys, indexing="ij")
    feats = [yy, xx]
    for f in range(4):
        wf = (2.0 ** f) * np.pi
        feats += [np.sin(wf * yy), np.cos(wf * yy),
                  np.sin(wf * xx), np.cos(wf * xx)]
    return np.stack(feats, 0).reshape(FCH, hp * hp).T.astype(np.float32)


_POSP = {}
for _hp, _N, _Npad in ((14, 784, 1024), (28, 3136, 3328)):
    _pf = np.tile(_posfeat_np(_hp), (BATCH, 1))
    _pp = np.zeros((_Npad, 128), np.float32)
    _pp[:_N, :FCH] = _pf
    _pp[:_N, FCH] = 1.0
    _POSP[_hp] = _pp


def _patch_adj_np(ps):
    A = np.zeros((9, ps * ps, ps * ps), np.float32)
    for pi in range(ps):
        for pj in range(ps):
            for di in (-1, 0, 1):
                for dj in (-1, 0, 1):
                    qi, qj = pi + di, pj + dj
                    if 0 <= qi < ps and 0 <= qj < ps:
                        k = (di + 1) * 3 + (dj + 1)
                        A[k, pi * ps + pj, qi * ps + qj] = 1.0
    return A


_ADJ = {ps: _patch_adj_np(ps) for ps in (2, 4)}
_AROW = np.zeros((3, 16, 16), np.float32)
for _pj in range(16):
    for _dj in (-1, 0, 1):
        _q = _pj + _dj
        if 0 <= _q < 16:
            _AROW[_dj + 1, _pj, _q] = 1.0


def _build_M_std(W8, ps, C):
    A = _ADJ[ps]
    Wk = jnp.transpose(W8, (0, 3, 4, 2, 1)).reshape(NE, 9, C, C)
    M = jnp.einsum("kpq,ekio->eqipo", jnp.asarray(A), Wk)
    return M.reshape(NE, ps * ps * C, ps * ps * C)


def _build_M_row(W8):
    Wk = jnp.transpose(W8, (0, 3, 4, 2, 1))
    M = jnp.einsum("jpq,edjio->edqipo", jnp.asarray(_AROW), Wk)
    return M.reshape(NE, 3 * 512, 512)


_NC, _NS = 2, 16
_NW = _NC * _NS


def _sc_gather(table, idx):
    V, D = table.shape
    B = idx.shape[0]
    assert B % _NW == 0 and D % 16 == 0
    rpw = B // _NW
    ch = min(rpw, 128)
    while ch > 0 and not (rpw % ch == 0 and ch * D * 4 <= 440_000):
        ch -= 8
    assert ch > 0
    nch = rpw // ch
    mesh = plsc.VectorSubcoreMesh(core_axis_name="c", subcore_axis_name="s")

    @functools.partial(
        pl.kernel, mesh=mesh,
        out_type=jax.ShapeDtypeStruct((B, D), table.dtype),
        scratch_types=[pltpu.VMEM((ch,), jnp.int32),
                       pltpu.VMEM((ch, D), table.dtype),
                       pltpu.SemaphoreType.DMA])
    def k(tab, ix, out, ixv, rows, sem):
        wid = lax.axis_index("s") * _NC + lax.axis_index("c")
        base = wid * rpw
        for c in range(nch):
            off = base + c * ch
            pltpu.sync_copy(ix.at[pl.ds(off, ch)], ixv)
            pltpu.async_copy(tab.at[ixv], rows, sem).wait()
            pltpu.sync_copy(rows, out.at[pl.ds(off, ch)])

    return k(table, idx)


def _stem_call(xt, xm, xb, w):
    B, RC = BATCH, 16
    grid = (B, 224 // RC)
    sp = pl.BlockSpec((1, RC, 232, 3), lambda b, j: (b, j, 0, 0))

    def kfn(t_r, m_r, b_r, w_r, o_r):
        cols = []
        for p in (t_r, m_r, b_r):
            for dj in range(3):
                cols.append(p[0, :, dj:dj + 224, :].reshape(RC * 224, 3))
        ic = jnp.concatenate(
            cols + [jnp.zeros((RC * 224, 5), F32)], axis=1)
        o_r[...] = _silu(_dot(ic, w_r[...]))[None]

    out = pl.pallas_call(
        kfn, grid=grid,
        in_specs=[sp, sp, sp, pl.BlockSpec((32, 32), lambda b, j: (0, 0))],
        out_specs=pl.BlockSpec((1, RC * 224, 32), lambda b, j: (b, j, 0)),
        out_shape=jax.ShapeDtypeStruct((B, 224 * 224, 32), F32),
    )(xt, xm, xb, w)
    return out.reshape(B, 224, 224, 32)


def _stem_layer(x, stem_w):
    xp = jnp.pad(x, ((0, 0), (1, 1), (1, 7), (0, 0)))
    w27 = jnp.transpose(stem_w, (2, 3, 1, 0)).reshape(27, 32)
    w27p = jnp.concatenate([w27, jnp.zeros((5, 32), F32)], axis=0)
    return _stem_call(xp[:, 0:224], xp[:, 1:225], xp[:, 2:226], w27p)


def _down_call(planes, wm, wsm, Hout, Wout, Cin, Co, RC, WE, WO):
    B = BATCH
    grid = (B, Hout // RC)
    se = pl.BlockSpec((1, RC, WE, Cin), lambda b, j: (b, j, 0, 0))
    so = pl.BlockSpec((1, RC, WO, Cin), lambda b, j: (b, j, 0, 0))

    def kfn(ae, ao, be, bo, ce, co_, wm_r, ws_r, o_r):
        cols = []
        for pe, po in ((ae, ao), (be, bo), (ce, co_)):
            cols.append(po[0, :, 0:Wout, :].reshape(RC * Wout, Cin))
            cols.append(pe[0, :, 0:Wout, :].reshape(RC * Wout, Cin))
            cols.append(po[0, :, 1:Wout + 1, :].reshape(RC * Wout, Cin))
        ic = jnp.concatenate(cols, axis=1)
        y = _silu(_dot(ic, wm_r[...])) + _dot(cols[4], ws_r[...])
        o_r[...] = y[None]

    out = pl.pallas_call(
        kfn, grid=grid,
        in_specs=[se, so, se, so, se, so,
                  pl.BlockSpec((9 * Cin, Co), lambda b, j: (0, 0)),
                  pl.BlockSpec((Cin, Co), lambda b, j: (0, 0))],
        out_specs=pl.BlockSpec((1, RC * Wout, Co), lambda b, j: (b, j, 0)),
        out_shape=jax.ShapeDtypeStruct((B, Hout * Wout, Co), F32),
    )(*planes, wm, wsm)
    return out.reshape(B, Hout, Wout, Co)


def _down_layer(img, w, ws):
    B, H, W_, Cin = img.shape
    Hout, Wout = H // 2, W_ // 2
    Co = w.shape[0]
    WE = ((Wout + 7) // 8) * 8
    WO = ((Wout + 8) // 8) * 8
    xe, xo = img[:, 0::2], img[:, 1::2]
    pa = jnp.concatenate(
        [jnp.zeros((B, 1, W_, Cin), F32), xo[:, :-1]], axis=1)
    planes = []
    for P in (pa, xe, xo):
        pe = P[:, :, 0::2, :]
        po = jnp.concatenate(
            [jnp.zeros((B, Hout, 1, Cin), F32), P[:, :, 1::2, :]], axis=2)
        planes.append(jnp.pad(pe, ((0, 0), (0, 0), (0, WE - Wout), (0, 0))))
        planes.append(jnp.pad(po, ((0, 0), (0, 0), (0, WO - Wout - 1), (0, 0))))
    wm = jnp.transpose(w, (2, 3, 1, 0)).reshape(9 * Cin, Co)
    wsm = ws[:, :, 0, 0].T
    RC = 28
    return _down_call(planes, wm, wsm, Hout, Wout, Cin, Co, RC, WE, WO)


def _router_call(x3, posp, gx, gp, cfg):
    N, Npad, Ccap, STRIDE = cfg["N"], cfg["Npad"], cfg["Ccap"], cfg["STRIDE"]
    S = NE * STRIDE
    T = 128
    two = cfg["ps"] == 16
    grid = (Npad // T,)
    if two:
        x_spec = pl.BlockSpec((T, 16, 512), lambda i: (i, 0, 0))
        dlx = 512
    else:
        D = cfg["ps"] ** 2 * cfg["C"]
        x_spec = pl.BlockSpec((T, D), lambda i: (i, 0))
        dlx = D

    def kfn(x_r, pp_r, gx_r, gp_r, meta_r, coef_r, slots_r, cnt):
        i = pl.program_id(0)

        @pl.when(i == 0)
        def _init():
            cnt[...] = jnp.zeros_like(cnt)
            slots_r[...] = jnp.zeros_like(slots_r)

        xb = x_r[...]
        xs = jnp.sum(xb, axis=1) if two else xb
        logits = _dot(xs, gx_r[...]) + _dot(pp_r[...], gp_r[...])
        lg = logits[:, 0:NE]
        mx = jnp.max(lg, axis=1, keepdims=True)
        ex = jnp.exp(lg - mx)
        probs = ex / jnp.sum(ex, axis=1, keepdims=True)
        pmax = jnp.max(probs, axis=1, keepdims=True)
        lane = lax.broadcasted_iota(F32, (T, NE), 1)
        t1 = jnp.min(jnp.where(probs >= pmax, lane, 1e9), axis=1,
                     keepdims=True)
        row = i * T + lax.broadcasted_iota(F32, (T, 1), 0)
        valid = row < N
        onehot = jnp.where((lane == t1) & valid, 1.0, 0.0)
        ri = lax.broadcasted_iota(F32, (T, T), 0)
        ci_ = lax.broadcasted_iota(F32, (T, T), 1)
        tri = jnp.where(ri >= ci_, 1.0, 0.0)
        csum = _dot(tri, onehot)
        prev = cnt[0:1, 0:NE]
        ppos = jnp.sum((csum + prev) * onehot, axis=1, keepdims=True) - 1.0
        keep = (ppos < Ccap) & valid
        coefv = jnp.where(keep, pmax, 0.0)
        sp_ = jnp.clip(ppos, 0.0, Ccap - 1.0)
        slotid = jnp.where(valid, t1 * STRIDE + sp_, 0.0)
        meta_r[...] = jnp.broadcast_to(slotid, (T, NE))
        coef_r[...] = jnp.broadcast_to(coefv, (T, NE))
        sid = lax.broadcasted_iota(F32, (T, S), 1)
        keepf = jnp.where(keep, 1.0, 0.0)
        eq = jnp.where(sid == slotid, 1.0, 0.0) * keepf
        contrib = jnp.sum(eq * row, axis=0, keepdims=True)
        slots_r[...] = slots_r[...] + jnp.broadcast_to(contrib, (NE, S))
        cnt[0:1, 0:NE] = prev + jnp.sum(onehot, axis=0, keepdims=True)

    return pl.pallas_call(
        kfn, grid=grid,
        in_specs=[x_spec,
                  pl.BlockSpec((T, 128), lambda i: (i, 0)),
                  pl.BlockSpec((dlx, 128), lambda i: (0, 0)),
                  pl.BlockSpec((128, 128), lambda i: (0, 0))],
        out_specs=[pl.BlockSpec((T, NE), lambda i: (i, 0)),
                   pl.BlockSpec((T, NE), lambda i: (i, 0)),
                   pl.BlockSpec((NE, S), lambda i: (0, 0))],
        out_shape=[jax.ShapeDtypeStruct((Npad, NE), F32),
                   jax.ShapeDtypeStruct((Npad, NE), F32),
                   jax.ShapeDtypeStruct((NE, S), F32)],
        scratch_shapes=[pltpu.VMEM((8, 128), F32)],
    )(x3, posp, gx, gp)


def _eka_call(xd3, M):
    S = xd3.shape[0]
    STRIDE = S // NE

    def kfn(x_r, m_r, o_r):
        xb = x_r[...]
        mm = m_r[0]
        z = jnp.zeros((STRIDE, 1, 512), F32)
        pb = jnp.concatenate([z, xb, z], axis=1)
        for i in range(16):
            xc = jnp.concatenate(
                [pb[:, i, :], pb[:, i + 1, :], pb[:, i + 2, :]], axis=1)
            o_r[:, i, :] = _silu(_dot(xc, mm))

    return pl.pallas_call(
        kfn, grid=(NE,),
        in_specs=[pl.BlockSpec((STRIDE, 16, 512), lambda e: (e, 0, 0)),
                  pl.BlockSpec((1, 1536, 512), lambda e: (e, 0, 0))],
        out_specs=pl.BlockSpec((STRIDE, 16, 512), lambda e: (e, 0, 0)),
        out_shape=jax.ShapeDtypeStruct((S, 16, 512), F32),
    )(xd3, M)


def _ekb_call(xd, M):
    S, D = xd.shape
    STRIDE = S // NE

    def kfn(x_r, m_r, o_r):
        o_r[...] = _silu(_dot(x_r[...], m_r[0]))

    return pl.pallas_call(
        kfn, grid=(NE,),
        in_specs=[pl.BlockSpec((STRIDE, D), lambda e: (e, 0)),
                  pl.BlockSpec((1, D, D), lambda e: (e, 0, 0))],
        out_specs=pl.BlockSpec((STRIDE, D), lambda e: (e, 0)),
        out_shape=jax.ShapeDtypeStruct((S, D), F32),
    )(xd, M)


def _combine_call(tok, yg, coef, gam8):
    Npad, D = tok.shape
    T = 128

    def kfn(x_r, y_r, c_r, g_r, o_r):
        o_r[...] = x_r[...] + g_r[0:1, :] * c_r[:, 0:1] * y_r[...]

    return pl.pallas_call(
        kfn, grid=(Npad // T,),
        in_specs=[pl.BlockSpec((T, D), lambda i: (i, 0)),
                  pl.BlockSpec((T, D), lambda i: (i, 0)),
                  pl.BlockSpec((T, NE), lambda i: (i, 0)),
                  pl.BlockSpec((NE, D), lambda i: (0, 0))],
        out_specs=pl.BlockSpec((T, D), lambda i: (i, 0)),
        out_shape=jax.ShapeDtypeStruct((Npad, D), F32),
    )(tok, yg, coef, gam8)


def _pce_layer(tok, l, params):
    cfg = _PCE[l]
    C, ps, N, Npad = cfg["C"], cfg["ps"], cfg["N"], cfg["Npad"]
    ps2 = ps * ps
    D = ps2 * C
    S = NE * cfg["STRIDE"]
    gW = params["pce%d_gateW" % l]
    gb = params["pce%d_gateb" % l]
    if ps == 16:
        gx = jnp.tile(gW[:C] / 256.0, (16, 1))
    else:
        gx = jnp.tile(gW[:C] / float(ps2), (ps2, 1))
    gx = jnp.pad(gx, ((0, 0), (0, 120)))
    gp = jnp.zeros((128, 128), F32)
    gp = gp.at[0:FCH, 0:NE].set(gW[C:]).at[FCH, 0:NE].set(gb)
    posp = jnp.asarray(_POSP[cfg["hp"]])
    x3 = tok.reshape(Npad, 16, 512) if ps == 16 else tok
    meta, coef, slots = _router_call(x3, posp, gx, gp, cfg)
    slot_tok = slots[0].astype(jnp.int32)
    tidx = meta[:, 0].astype(jnp.int32)
    xd = _sc_gather(tok, slot_tok)
    W8 = params["pce%d_experts" % l]
    if ps == 16:
        y = _eka_call(xd.reshape(S, 16, 512), _build_M_row(W8)).reshape(S, D)
    elif ps == 1:
        y = _ekb_call(xd, jnp.transpose(W8[:, :, :, 1, 1], (0, 2, 1)))
    else:
        y = _ekb_call(xd, _build_M_std(W8, ps, C))
    yg = _sc_gather(y, tidx)
    gamD = jnp.tile(params["pce%d_gamma" % l], (ps2,))
    gam8 = jnp.broadcast_to(gamD[None, :], (NE, D))
    return _combine_call(tok, yg, coef, gam8)


def _extract(img, ps, cfg):
    B, H, W_, C = img.shape
    hp = H // ps
    t = img.reshape(B, hp, ps, hp, ps, C).transpose(0, 1, 3, 2, 4, 5)
    t = t.reshape(B * hp * hp, ps * ps * C)
    return jnp.pad(t, ((0, cfg["Npad"] - B * hp * hp), (0, 0)))


def _reassemble(tok, ps, cfg, C):
    B = BATCH
    hp = cfg["hp"]
    t = tok[:cfg["N"]].reshape(B, hp, hp, ps, ps, C)
    t = t.transpose(0, 1, 3, 2, 4, 5).reshape(B, hp * ps, hp * ps, C)
    return t


def _head_call(hx, params):
    g = jnp.tile(params["head_ln_g"][None, :], (8, 1))
    b = jnp.tile(params["head_ln_b"][None, :], (8, 1))
    hb = jnp.tile(params["head_b"][None, :], (8, 1))
    W = params["head_W"]

    def kfn(x_r, g_r, b_r, w_r, hb_r, o_r):
        f = jnp.mean(x_r[...], axis=1)
        mu = jnp.mean(f, axis=1, keepdims=True)
        var = jnp.mean((f - mu) ** 2, axis=1, keepdims=True)
        fn = (f - mu) * lax.rsqrt(var + 1e-5) * g_r[0:1, :] + b_r[0:1, :]
        o = _dot(fn, w_r[...]) + hb_r[0:1, :]
        o_r[...] = jnp.concatenate([o, jnp.zeros((4, 1000), F32)], axis=0)

    out = pl.pallas_call(
        kfn, grid=(1,),
        in_specs=[pl.BlockSpec((4, 784, 256), lambda i: (0, 0, 0)),
                  pl.BlockSpec((8, 256), lambda i: (0, 0)),
                  pl.BlockSpec((8, 256), lambda i: (0, 0)),
                  pl.BlockSpec((256, 1000), lambda i: (0, 0)),
                  pl.BlockSpec((8, 1000), lambda i: (0, 0))],
        out_specs=pl.BlockSpec((8, 1000), lambda i: (0, 0)),
        out_shape=jax.ShapeDtypeStruct((8, 1000), F32),
    )(hx, g, b, W, hb)
    return out[:4]


def kernel(X, params):
    x = jnp.transpose(X, (0, 2, 3, 1)).astype(F32)
    img = _stem_layer(x, params["stem_w"])
    ps = 16
    tok = None
    for l in range(8):
        if l in (2, 4, 6):
            img = _down_layer(img, params["down%d_w" % l],
                              params["down%d_skip" % l])
            ps = ps // 4 if l == 2 else max(1, ps // 2)
        else:
            cfg = _PCE[l]
            if l == 0 or (l - 1) in (2, 4, 6):
                tok = _extract(img, ps, cfg)
            tok = _pce_layer(tok, l, params)
            if (l + 1) in (2, 4, 6) or l == 7:
                img = _reassemble(tok, ps, cfg, cfg["C"])
    hx = img.reshape(BATCH, 784, 256)
    return _head_call(hx, params)


# speedup vs baseline: 1.1950x; 1.1950x over previous
import functools
import math

import numpy as np
import jax
import jax.numpy as jnp
from jax import lax
from jax.experimental import pallas as pl
from jax.experimental.pallas import tpu as pltpu
from jax.experimental.pallas import tpu_sc as plsc

F32 = jnp.float32
NE = 8
FCH = 18
BATCH = 4

_PCE = {
    0: dict(C=32, ps=16, hp=14, N=784, Npad=1024, Ccap=123, STRIDE=128),
    1: dict(C=32, ps=16, hp=14, N=784, Npad=1024, Ccap=123, STRIDE=128),
    3: dict(C=64, ps=4, hp=28, N=3136, Npad=3328, Ccap=490, STRIDE=512),
    5: dict(C=128, ps=2, hp=28, N=3136, Npad=3328, Ccap=490, STRIDE=512),
    7: dict(C=256, ps=1, hp=28, N=3136, Npad=3328, Ccap=490, STRIDE=512),
}


def _silu(x):
    return x * jax.nn.sigmoid(x)


def _dot(a, b):
    return lax.dot_general(a, b, (((a.ndim - 1,), (0,)), ((), ())),
                           preferred_element_type=F32)


def _posfeat_np(hp):
    ys = (np.arange(hp) + 0.5) / hp
    yy, xx = np.meshgrid(ys, ys, indexing="ij")
    feats = [yy, xx]
    for f in range(4):
        wf = (2.0 ** f) * np.pi
        feats += [np.sin(wf * yy), np.cos(wf * yy),
                  np.sin(wf * xx), np.cos(wf * xx)]
    return np.stack(feats, 0).reshape(FCH, hp * hp).T.astype(np.float32)


_POSP = {}
for _hp, _N, _Npad in ((14, 784, 1024), (28, 3136, 3328)):
    _pf = np.tile(_posfeat_np(_hp), (BATCH, 1))
    _pp = np.zeros((_Npad, 128), np.float32)
    _pp[:_N, :FCH] = _pf
    _pp[:_N, FCH] = 1.0
    _POSP[_hp] = _pp


def _patch_adj_np(ps):
    A = np.zeros((9, ps * ps, ps * ps), np.float32)
    for pi in range(ps):
        for pj in range(ps):
            for di in (-1, 0, 1):
                for dj in (-1, 0, 1):
                    qi, qj = pi + di, pj + dj
                    if 0 <= qi < ps and 0 <= qj < ps:
                        k = (di + 1) * 3 + (dj + 1)
                        A[k, pi * ps + pj, qi * ps + qj] = 1.0
    return A


_ADJ = {ps: _patch_adj_np(ps) for ps in (2, 4)}
_AROW = np.zeros((3, 16, 16), np.float32)
for _pj in range(16):
    for _dj in (-1, 0, 1):
        _q = _pj + _dj
        if 0 <= _q < 16:
            _AROW[_dj + 1, _pj, _q] = 1.0


def _build_M_std(W8, ps, C):
    A = _ADJ[ps]
    Wk = jnp.transpose(W8, (0, 3, 4, 2, 1)).reshape(NE, 9, C, C)
    M = jnp.einsum("kpq,ekio->eqipo", jnp.asarray(A), Wk)
    return M.reshape(NE, ps * ps * C, ps * ps * C)


def _build_M_row(W8):
    Wk = jnp.transpose(W8, (0, 3, 4, 2, 1))
    M = jnp.einsum("jpq,edjio->edqipo", jnp.asarray(_AROW), Wk)
    return M.reshape(NE, 3 * 512, 512)


_NC, _NS = 2, 16
_NW = _NC * _NS


def _sc_gather(table, idx):
    V, D = table.shape
    B = idx.shape[0]
    assert B % _NW == 0 and D % 16 == 0
    rpw = B // _NW
    ch = min(rpw, 128)
    while ch > 0 and not (rpw % ch == 0 and ch * D * 4 <= 440_000):
        ch -= 8
    assert ch > 0
    nch = rpw // ch
    mesh = plsc.VectorSubcoreMesh(core_axis_name="c", subcore_axis_name="s")

    @functools.partial(
        pl.kernel, mesh=mesh,
        out_type=jax.ShapeDtypeStruct((B, D), table.dtype),
        scratch_types=[pltpu.VMEM((ch,), jnp.int32),
                       pltpu.VMEM((ch, D), table.dtype),
                       pltpu.SemaphoreType.DMA])
    def k(tab, ix, out, ixv, rows, sem):
        wid = lax.axis_index("s") * _NC + lax.axis_index("c")
        base = wid * rpw
        for c in range(nch):
            off = base + c * ch
            pltpu.sync_copy(ix.at[pl.ds(off, ch)], ixv)
            pltpu.async_copy(tab.at[ixv], rows, sem).wait()
            pltpu.sync_copy(rows, out.at[pl.ds(off, ch)])

    return k(table, idx)


def _stem_call(xt, xm, xb, w):
    B, RC = BATCH, 16
    grid = (B, 224 // RC)
    sp = pl.BlockSpec((1, RC, 232, 3), lambda b, j: (b, j, 0, 0))

    def kfn(t_r, m_r, b_r, w_r, o_r):
        cols = []
        for p in (t_r, m_r, b_r):
            for dj in range(3):
                cols.append(p[0, :, dj:dj + 224, :].reshape(RC * 224, 3))
        ic = jnp.concatenate(
            cols + [jnp.zeros((RC * 224, 5), F32)], axis=1)
        o_r[...] = _silu(_dot(ic, w_r[...]))[None]

    out = pl.pallas_call(
        kfn, grid=grid,
        in_specs=[sp, sp, sp, pl.BlockSpec((32, 32), lambda b, j: (0, 0))],
        out_specs=pl.BlockSpec((1, RC * 224, 32), lambda b, j: (b, j, 0)),
        out_shape=jax.ShapeDtypeStruct((B, 224 * 224, 32), F32),
    )(xt, xm, xb, w)
    return out.reshape(B, 224, 224, 32)


def _stem_layer(x, stem_w):
    xp = jnp.pad(x, ((0, 0), (1, 1), (1, 7), (0, 0)))
    w27 = jnp.transpose(stem_w, (2, 3, 1, 0)).reshape(27, 32)
    w27p = jnp.concatenate([w27, jnp.zeros((5, 32), F32)], axis=0)
    return _stem_call(xp[:, 0:224], xp[:, 1:225], xp[:, 2:226], w27p)


def _down_call(planes, wm, wsm, Hout, Wout, Cin, Co, RC, WE, WO):
    B = BATCH
    grid = (B, Hout // RC)
    se = pl.BlockSpec((1, RC, WE, Cin), lambda b, j: (b, j, 0, 0))
    so = pl.BlockSpec((1, RC, WO, Cin), lambda b, j: (b, j, 0, 0))

    def kfn(ae, ao, be, bo, ce, co_, wm_r, ws_r, o_r):
        cols = []
        for pe, po in ((ae, ao), (be, bo), (ce, co_)):
            cols.append(po[0, :, 0:Wout, :].reshape(RC * Wout, Cin))
            cols.append(pe[0, :, 0:Wout, :].reshape(RC * Wout, Cin))
            cols.append(po[0, :, 1:Wout + 1, :].reshape(RC * Wout, Cin))
        ic = jnp.concatenate(cols, axis=1)
        y = _silu(_dot(ic, wm_r[...])) + _dot(cols[4], ws_r[...])
        o_r[...] = y[None]

    out = pl.pallas_call(
        kfn, grid=grid,
        in_specs=[se, so, se, so, se, so,
                  pl.BlockSpec((9 * Cin, Co), lambda b, j: (0, 0)),
                  pl.BlockSpec((Cin, Co), lambda b, j: (0, 0))],
        out_specs=pl.BlockSpec((1, RC * Wout, Co), lambda b, j: (b, j, 0)),
        out_shape=jax.ShapeDtypeStruct((B, Hout * Wout, Co), F32),
    )(*planes, wm, wsm)
    return out.reshape(B, Hout, Wout, Co)


def _down_layer(img, w, ws):
    B, H, W_, Cin = img.shape
    Hout, Wout = H // 2, W_ // 2
    Co = w.shape[0]
    WE = ((Wout + 7) // 8) * 8
    WO = ((Wout + 8) // 8) * 8
    xe, xo = img[:, 0::2], img[:, 1::2]
    pa = jnp.concatenate(
        [jnp.zeros((B, 1, W_, Cin), F32), xo[:, :-1]], axis=1)
    planes = []
    for P in (pa, xe, xo):
        pe = P[:, :, 0::2, :]
        po = jnp.concatenate(
            [jnp.zeros((B, Hout, 1, Cin), F32), P[:, :, 1::2, :]], axis=2)
        planes.append(jnp.pad(pe, ((0, 0), (0, 0), (0, WE - Wout), (0, 0))))
        planes.append(jnp.pad(po, ((0, 0), (0, 0), (0, WO - Wout - 1), (0, 0))))
    wm = jnp.transpose(w, (2, 3, 1, 0)).reshape(9 * Cin, Co)
    wsm = ws[:, :, 0, 0].T
    RC = 28
    return _down_call(planes, wm, wsm, Hout, Wout, Cin, Co, RC, WE, WO)


def _router_call(x3, posp, gx, gp, cfg):
    N, Npad, Ccap, STRIDE = cfg["N"], cfg["Npad"], cfg["Ccap"], cfg["STRIDE"]
    S = NE * STRIDE
    T = 128
    two = cfg["ps"] == 16
    grid = (Npad // T,)
    if two:
        x_spec = pl.BlockSpec((T, 16, 512), lambda i: (i, 0, 0))
        dlx = 512
    else:
        D = cfg["ps"] ** 2 * cfg["C"]
        x_spec = pl.BlockSpec((T, D), lambda i: (i, 0))
        dlx = D

    def kfn(x_r, pp_r, gx_r, gp_r, meta_r, coef_r, slots_r, cnt):
        i = pl.program_id(0)

        @pl.when(i == 0)
        def _init():
            cnt[...] = jnp.zeros_like(cnt)
            slots_r[...] = jnp.zeros_like(slots_r)

        xb = x_r[...]
        xs = jnp.sum(xb, axis=1) if two else xb
        logits = _dot(xs, gx_r[...]) + _dot(pp_r[...], gp_r[...])
        lg = logits[:, 0:NE]
        mx = jnp.max(lg, axis=1, keepdims=True)
        ex = jnp.exp(lg - mx)
        probs = ex / jnp.sum(ex, axis=1, keepdims=True)
        pmax = jnp.max(probs, axis=1, keepdims=True)
        lane = lax.broadcasted_iota(jnp.int32, (T, NE), 1).astype(F32)
        t1 = jnp.min(jnp.where(probs >= pmax, lane, 1e9), axis=1,
                     keepdims=True)
        row = (i * T + lax.broadcasted_iota(jnp.int32, (T, 1), 0)).astype(F32)
        valid = row < N
        onehot = jnp.where((lane == t1) & valid, 1.0, 0.0)
        ri = lax.broadcasted_iota(jnp.int32, (T, T), 0)
        ci_ = lax.broadcasted_iota(jnp.int32, (T, T), 1)
        tri = jnp.where(ri >= ci_, 1.0, 0.0)
        csum = _dot(tri, onehot)
        prev = cnt[0:1, 0:NE]
        ppos = jnp.sum((csum + prev) * onehot, axis=1, keepdims=True) - 1.0
        keep = (ppos < Ccap) & valid
        coefv = jnp.where(keep, pmax, 0.0)
        sp_ = jnp.clip(ppos, 0.0, Ccap - 1.0)
        slotid = jnp.where(valid, t1 * STRIDE + sp_, 0.0)
        meta_r[...] = jnp.broadcast_to(slotid, (T, NE))
        coef_r[...] = jnp.broadcast_to(coefv, (T, NE))
        sid = lax.broadcasted_iota(jnp.int32, (T, S), 1).astype(F32)
        keepf = jnp.where(keep, 1.0, 0.0)
        eq = jnp.where(sid == slotid, 1.0, 0.0) * keepf
        contrib = jnp.sum(eq * row, axis=0, keepdims=True)
        slots_r[...] = slots_r[...] + jnp.broadcast_to(contrib, (NE, S))
        cnt[0:1, 0:NE] = prev + jnp.sum(onehot, axis=0, keepdims=True)

    return pl.pallas_call(
        kfn, grid=grid,
        in_specs=[x_spec,
                  pl.BlockSpec((T, 128), lambda i: (i, 0)),
                  pl.BlockSpec((dlx, 128), lambda i: (0, 0)),
                  pl.BlockSpec((128, 128), lambda i: (0, 0))],
        out_specs=[pl.BlockSpec((T, NE), lambda i: (i, 0)),
                   pl.BlockSpec((T, NE), lambda i: (i, 0)),
                   pl.BlockSpec((NE, S), lambda i: (0, 0))],
        out_shape=[jax.ShapeDtypeStruct((Npad, NE), F32),
                   jax.ShapeDtypeStruct((Npad, NE), F32),
                   jax.ShapeDtypeStruct((NE, S), F32)],
        scratch_shapes=[pltpu.VMEM((8, 128), F32)],
    )(x3, posp, gx, gp)


def _eka_call(xd3, M):
    S = xd3.shape[0]
    STRIDE = S // NE

    def kfn(x_r, m_r, o_r):
        xb = x_r[...]
        mm = m_r[0]
        z = jnp.zeros((STRIDE, 1, 512), F32)
        pb = jnp.concatenate([z, xb, z], axis=1)
        for i in range(16):
            xc = jnp.concatenate(
                [pb[:, i, :], pb[:, i + 1, :], pb[:, i + 2, :]], axis=1)
            o_r[:, i, :] = _silu(_dot(xc, mm))

    return pl.pallas_call(
        kfn, grid=(NE,),
        in_specs=[pl.BlockSpec((STRIDE, 16, 512), lambda e: (e, 0, 0)),
                  pl.BlockSpec((1, 1536, 512), lambda e: (e, 0, 0))],
        out_specs=pl.BlockSpec((STRIDE, 16, 512), lambda e: (e, 0, 0)),
        out_shape=jax.ShapeDtypeStruct((S, 16, 512), F32),
    )(xd3, M)


def _ekb_call(xd, M):
    S, D = xd.shape
    STRIDE = S // NE

    def kfn(x_r, m_r, o_r):
        o_r[...] = _silu(_dot(x_r[...], m_r[0]))

    return pl.pallas_call(
        kfn, grid=(NE,),
        in_specs=[pl.BlockSpec((STRIDE, D), lambda e: (e, 0)),
                  pl.BlockSpec((1, D, D), lambda e: (e, 0, 0))],
        out_specs=pl.BlockSpec((STRIDE, D), lambda e: (e, 0)),
        out_shape=jax.ShapeDtypeStruct((S, D), F32),
    )(xd, M)


def _combine_call(tok, yg, coef, gam8):
    Npad, D = tok.shape
    T = 128

    def kfn(x_r, y_r, c_r, g_r, o_r):
        o_r[...] = x_r[...] + g_r[0:1, :] * c_r[:, 0:1] * y_r[...]

    return pl.pallas_call(
        kfn, grid=(Npad // T,),
        in_specs=[pl.BlockSpec((T, D), lambda i: (i, 0)),
                  pl.BlockSpec((T, D), lambda i: (i, 0)),
                  pl.BlockSpec((T, NE), lambda i: (i, 0)),
                  pl.BlockSpec((NE, D), lambda i: (0, 0))],
        out_specs=pl.BlockSpec((T, D), lambda i: (i, 0)),
        out_shape=jax.ShapeDtypeStruct((Npad, D), F32),
    )(tok, yg, coef, gam8)


def _pce_layer(tok, l, params):
    cfg = _PCE[l]
    C, ps, N, Npad = cfg["C"], cfg["ps"], cfg["N"], cfg["Npad"]
    ps2 = ps * ps
    D = ps2 * C
    S = NE * cfg["STRIDE"]
    gW = params["pce%d_gateW" % l]
    gb = params["pce%d_gateb" % l]
    if ps == 16:
        gx = jnp.tile(gW[:C] / 256.0, (16, 1))
    else:
        gx = jnp.tile(gW[:C] / float(ps2), (ps2, 1))
    gx = jnp.pad(gx, ((0, 0), (0, 120)))
    gp = jnp.zeros((128, 128), F32)
    gp = gp.at[0:FCH, 0:NE].set(gW[C:]).at[FCH, 0:NE].set(gb)
    posp = jnp.asarray(_POSP[cfg["hp"]])
    x3 = tok.reshape(Npad, 16, 512) if ps == 16 else tok
    meta, coef, slots = _router_call(x3, posp, gx, gp, cfg)
    slot_tok = slots[0].astype(jnp.int32)
    tidx = meta[:, 0].astype(jnp.int32)
    xd = _sc_gather(tok, slot_tok)
    W8 = params["pce%d_experts" % l]
    if ps == 16:
        y = _eka_call(xd.reshape(S, 16, 512), _build_M_row(W8)).reshape(S, D)
    elif ps == 1:
        y = _ekb_call(xd, jnp.transpose(W8[:, :, :, 1, 1], (0, 2, 1)))
    else:
        y = _ekb_call(xd, _build_M_std(W8, ps, C))
    yg = _sc_gather(y, tidx)
    gamD = jnp.tile(params["pce%d_gamma" % l], (ps2,))
    gam8 = jnp.broadcast_to(gamD[None, :], (NE, D))
    return _combine_call(tok, yg, coef, gam8)


def _extract(img, ps, cfg):
    B, H, W_, C = img.shape
    hp = H // ps
    t = img.reshape(B, hp, ps, hp, ps, C).transpose(0, 1, 3, 2, 4, 5)
    t = t.reshape(B * hp * hp, ps * ps * C)
    return jnp.pad(t, ((0, cfg["Npad"] - B * hp * hp), (0, 0)))


def _reassemble(tok, ps, cfg, C):
    B = BATCH
    hp = cfg["hp"]
    t = tok[:cfg["N"]].reshape(B, hp, hp, ps, ps, C)
    t = t.transpose(0, 1, 3, 2, 4, 5).reshape(B, hp * ps, hp * ps, C)
    return t


def _head_call(hx, params):
    g = jnp.tile(params["head_ln_g"][None, :], (8, 1))
    b = jnp.tile(params["head_ln_b"][None, :], (8, 1))
    hb = jnp.tile(params["head_b"][None, :], (8, 1))
    W = params["head_W"]

    def kfn(x_r, g_r, b_r, w_r, hb_r, o_r):
        f = jnp.mean(x_r[...], axis=1)
        mu = jnp.mean(f, axis=1, keepdims=True)
        var = jnp.mean((f - mu) ** 2, axis=1, keepdims=True)
        fn = (f - mu) * lax.rsqrt(var + 1e-5) * g_r[0:1, :] + b_r[0:1, :]
        o = _dot(fn, w_r[...]) + hb_r[0:1, :]
        o_r[...] = jnp.concatenate([o, jnp.zeros((4, 1000), F32)], axis=0)

    out = pl.pallas_call(
        kfn, grid=(1,),
        in_specs=[pl.BlockSpec((4, 784, 256), lambda i: (0, 0, 0)),
                  pl.BlockSpec((8, 256), lambda i: (0, 0)),
                  pl.BlockSpec((8, 256), lambda i: (0, 0)),
                  pl.BlockSpec((256, 1000), lambda i: (0, 0)),
                  pl.BlockSpec((8, 1000), lambda i: (0, 0))],
        out_specs=pl.BlockSpec((8, 1000), lambda i: (0, 0)),
        out_shape=jax.ShapeDtypeStruct((8, 1000), F32),
    )(hx, g, b, W, hb)
    return out[:4]


def kernel(X, params):
    x = jnp.transpose(X, (0, 2, 3, 1)).astype(F32)
    img = _stem_layer(x, params["stem_w"])
    ps = 16
    tok = None
    for l in range(8):
        if l in (2, 4, 6):
            img = _down_layer(img, params["down%d_w" % l],
                              params["down%d_skip" % l])
            ps = ps // 4 if l == 2 else max(1, ps // 2)
        else:
            cfg = _PCE[l]
            if l == 0 or (l - 1) in (2, 4, 6):
                tok = _extract(img, ps, cfg)
            tok = _pce_layer(tok, l, params)
            if (l + 1) in (2, 4, 6) or l == 7:
                img = _reassemble(tok, ps, cfg, cfg["C"])
    hx = img.reshape(BATCH, 784, 256)
    return _head_call(hx, params)


# speedup vs baseline: 1.7196x; 1.4390x over previous
import functools
import math

import numpy as np
import jax
import jax.numpy as jnp
from jax import lax
from jax.experimental import pallas as pl
from jax.experimental.pallas import tpu as pltpu
from jax.experimental.pallas import tpu_sc as plsc

F32 = jnp.float32
NE = 8
FCH = 18
BATCH = 4

_PCE = {
    0: dict(C=32, ps=16, hp=14, N=784, Npad=1024, Ccap=123, STRIDE=128),
    1: dict(C=32, ps=16, hp=14, N=784, Npad=1024, Ccap=123, STRIDE=128),
    3: dict(C=64, ps=4, hp=28, N=3136, Npad=3328, Ccap=490, STRIDE=512),
    5: dict(C=128, ps=2, hp=28, N=3136, Npad=3328, Ccap=490, STRIDE=512),
    7: dict(C=256, ps=1, hp=28, N=3136, Npad=3328, Ccap=490, STRIDE=512),
}


def _silu(x):
    return x * jax.nn.sigmoid(x)


def _dot(a, b):
    return lax.dot_general(a, b, (((a.ndim - 1,), (0,)), ((), ())),
                           preferred_element_type=F32)


def _posfeat_np(hp):
    ys = (np.arange(hp) + 0.5) / hp
    yy, xx = np.meshgrid(ys, ys, indexing="ij")
    feats = [yy, xx]
    for f in range(4):
        wf = (2.0 ** f) * np.pi
        feats += [np.sin(wf * yy), np.cos(wf * yy),
                  np.sin(wf * xx), np.cos(wf * xx)]
    return np.stack(feats, 0).reshape(FCH, hp * hp).T.astype(np.float32)


_POSP = {}
for _hp, _N, _Npad in ((14, 784, 1024), (28, 3136, 3328)):
    _pf = np.tile(_posfeat_np(_hp), (BATCH, 1))
    _pp = np.zeros((_Npad, 128), np.float32)
    _pp[:_N, :FCH] = _pf
    _pp[:_N, FCH] = 1.0
    _POSP[_hp] = _pp


def _patch_adj_np(ps):
    A = np.zeros((9, ps * ps, ps * ps), np.float32)
    for pi in range(ps):
        for pj in range(ps):
            for di in (-1, 0, 1):
                for dj in (-1, 0, 1):
                    qi, qj = pi + di, pj + dj
                    if 0 <= qi < ps and 0 <= qj < ps:
                        k = (di + 1) * 3 + (dj + 1)
                        A[k, pi * ps + pj, qi * ps + qj] = 1.0
    return A


_ADJ = {ps: _patch_adj_np(ps) for ps in (2, 4)}
_AROW = np.zeros((3, 16, 16), np.float32)
for _pj in range(16):
    for _dj in (-1, 0, 1):
        _q = _pj + _dj
        if 0 <= _q < 16:
            _AROW[_dj + 1, _pj, _q] = 1.0


def _build_M_std(W8, ps, C):
    A = _ADJ[ps]
    Wk = jnp.transpose(W8, (0, 3, 4, 2, 1)).reshape(NE, 9, C, C)
    M = jnp.einsum("kpq,ekio->eqipo", jnp.asarray(A), Wk)
    return M.reshape(NE, ps * ps * C, ps * ps * C)


def _build_M_row(W8):
    Wk = jnp.transpose(W8, (0, 3, 4, 2, 1))
    M = jnp.einsum("jpq,edjio->edqipo", jnp.asarray(_AROW), Wk)
    return M.reshape(NE, 3 * 512, 512)


_NC, _NS = 2, 16
_NW = _NC * _NS


def _sc_gather(table, idx):
    V, D = table.shape
    B = idx.shape[0]
    assert B % _NW == 0 and D % 16 == 0
    rpw = B // _NW
    ch = min(rpw, 128)
    while ch > 0 and not (rpw % ch == 0 and ch * D * 4 <= 440_000):
        ch -= 8
    assert ch > 0
    nch = rpw // ch
    mesh = plsc.VectorSubcoreMesh(core_axis_name="c", subcore_axis_name="s")

    @functools.partial(
        pl.kernel, mesh=mesh,
        out_type=jax.ShapeDtypeStruct((B, D), table.dtype),
        scratch_types=[pltpu.VMEM((ch,), jnp.int32),
                       pltpu.VMEM((ch, D), table.dtype),
                       pltpu.SemaphoreType.DMA])
    def k(tab, ix, out, ixv, rows, sem):
        wid = lax.axis_index("s") * _NC + lax.axis_index("c")
        base = wid * rpw
        for c in range(nch):
            off = base + c * ch
            pltpu.sync_copy(ix.at[pl.ds(off, ch)], ixv)
            pltpu.async_copy(tab.at[ixv], rows, sem).wait()
            pltpu.sync_copy(rows, out.at[pl.ds(off, ch)])

    return k(table, idx)


def _stem_call(xp, w):
    B, RC = BATCH, 16
    nb = 224 // RC
    grid = (B, nb)
    sa = pl.BlockSpec((1, RC, 232, 3), lambda b, j: (b, j, 0, 0))
    sb = pl.BlockSpec((1, RC, 232, 3), lambda b, j: (b, j + 1, 0, 0))

    def kfn(a_r, b_r, w_r, o_r):
        rows = jnp.concatenate([a_r[0], b_r[0, 0:2]], axis=0)
        cols = []
        for di in range(3):
            p = rows[di:di + RC]
            for dj in range(3):
                cols.append(p[:, dj:dj + 224, :].reshape(RC * 224, 3))
        ic = jnp.concatenate(
            cols + [jnp.zeros((RC * 224, 5), F32)], axis=1)
        o_r[...] = _silu(_dot(ic, w_r[...]))[None]

    out = pl.pallas_call(
        kfn, grid=grid,
        in_specs=[sa, sb, pl.BlockSpec((32, 32), lambda b, j: (0, 0))],
        out_specs=pl.BlockSpec((1, RC * 224, 32), lambda b, j: (b, j, 0)),
        out_shape=jax.ShapeDtypeStruct((B, 224 * 224, 32), F32),
    )(xp, xp, w)
    return out.reshape(B, 224, 224, 32)


def _stem_layer(x, stem_w):
    xp = jnp.pad(x, ((0, 0), (1, 15), (1, 7), (0, 0)))
    w27 = jnp.transpose(stem_w, (2, 3, 1, 0)).reshape(27, 32)
    w27p = jnp.concatenate([w27, jnp.zeros((5, 32), F32)], axis=0)
    return _stem_call(xp, w27p)


def _down_call(xpv, wm, wsm, Hout, Wout, Cin, Co, RC):
    B = BATCH
    nb = Hout // RC
    grid = (B, nb)
    C2 = 2 * Cin
    sa = pl.BlockSpec((1, 2 * RC, Wout, C2), lambda b, j: (b, j, 0, 0))
    sb = pl.BlockSpec((1, 2 * RC, Wout, C2), lambda b, j: (b, j + 1, 0, 0))

    def kfn(a_r, b_r, wm_r, ws_r, o_r):
        rows = jnp.concatenate([a_r[0], b_r[0, 0:2]], axis=0)
        cols = []
        skip = None
        for di in range(3):
            rdi = rows[di:di + 2 * RC].reshape(RC, 2, Wout, C2)[:, 0]
            ev = rdi[:, :, 0:Cin]
            od = rdi[:, :, Cin:C2]
            om = jnp.concatenate(
                [jnp.zeros((RC, 1, Cin), F32), od[:, 0:Wout - 1, :]], axis=1)
            cols += [om.reshape(RC * Wout, Cin),
                     ev.reshape(RC * Wout, Cin),
                     od.reshape(RC * Wout, Cin)]
            if di == 1:
                skip = cols[4]
        ic = jnp.concatenate(cols, axis=1)
        y = _silu(_dot(ic, wm_r[...])) + _dot(skip, ws_r[...])
        o_r[...] = y[None]

    out = pl.pallas_call(
        kfn, grid=grid,
        in_specs=[sa, sb,
                  pl.BlockSpec((9 * Cin, Co), lambda b, j: (0, 0)),
                  pl.BlockSpec((Cin, Co), lambda b, j: (0, 0))],
        out_specs=pl.BlockSpec((1, RC * Wout, Co), lambda b, j: (b, j, 0)),
        out_shape=jax.ShapeDtypeStruct((B, Hout * Wout, Co), F32),
    )(xpv, xpv, wm, wsm)
    return out.reshape(B, Hout, Wout, Co)


def _down_layer(img, w, ws):
    B, H, W_, Cin = img.shape
    Hout, Wout = H // 2, W_ // 2
    Co = w.shape[0]
    RC = 28
    nb = Hout // RC
    hpad = (nb + 1) * 2 * RC
    xp = jnp.pad(img, ((0, 0), (1, hpad - H - 1), (0, 0), (0, 0)))
    xpv = xp.reshape(B, hpad, Wout, 2 * Cin)
    wm = jnp.transpose(w, (2, 3, 1, 0)).reshape(9 * Cin, Co)
    wsm = ws[:, :, 0, 0].T
    return _down_call(xpv, wm, wsm, Hout, Wout, Cin, Co, RC)


def _router_call(x3, posp, gx, gp, cfg):
    N, Npad, Ccap, STRIDE = cfg["N"], cfg["Npad"], cfg["Ccap"], cfg["STRIDE"]
    S = NE * STRIDE
    T = 128
    two = cfg["ps"] == 16
    grid = (Npad // T,)
    if two:
        x_spec = pl.BlockSpec((T, 16, 512), lambda i: (i, 0, 0))
        dlx = 512
    else:
        D = cfg["ps"] ** 2 * cfg["C"]
        x_spec = pl.BlockSpec((T, D), lambda i: (i, 0))
        dlx = D

    def kfn(x_r, pp_r, gx_r, gp_r, meta_r, coef_r, slots_r, cnt):
        i = pl.program_id(0)

        @pl.when(i == 0)
        def _init():
            cnt[...] = jnp.zeros_like(cnt)
            slots_r[...] = jnp.zeros_like(slots_r)

        xb = x_r[...]
        xs = jnp.sum(xb, axis=1) if two else xb
        logits = _dot(xs, gx_r[...]) + _dot(pp_r[...], gp_r[...])
        lg = logits[:, 0:NE]
        mx = jnp.max(lg, axis=1, keepdims=True)
        ex = jnp.exp(lg - mx)
        probs = ex / jnp.sum(ex, axis=1, keepdims=True)
        pmax = jnp.max(probs, axis=1, keepdims=True)
        lane = lax.broadcasted_iota(jnp.int32, (T, NE), 1).astype(F32)
        t1 = jnp.min(jnp.where(probs >= pmax, lane, 1e9), axis=1,
                     keepdims=True)
        row = (i * T + lax.broadcasted_iota(jnp.int32, (T, 1), 0)).astype(F32)
        valid = row < N
        onehot = jnp.where((lane == t1) & valid, 1.0, 0.0)
        ri = lax.broadcasted_iota(jnp.int32, (T, T), 0)
        ci_ = lax.broadcasted_iota(jnp.int32, (T, T), 1)
        tri = jnp.where(ri >= ci_, 1.0, 0.0)
        csum = _dot(tri, onehot)
        prev = cnt[0:1, 0:NE]
        ppos = jnp.sum((csum + prev) * onehot, axis=1, keepdims=True) - 1.0
        keep = (ppos < Ccap) & valid
        coefv = jnp.where(keep, pmax, 0.0)
        sp_ = jnp.clip(ppos, 0.0, Ccap - 1.0)
        slotid = jnp.where(valid, t1 * STRIDE + sp_, 0.0)
        meta_r[...] = jnp.broadcast_to(slotid, (T, NE))
        coef_r[...] = jnp.broadcast_to(coefv, (T, NE))
        sid = lax.broadcasted_iota(jnp.int32, (T, S), 1).astype(F32)
        keepf = jnp.where(keep, 1.0, 0.0)
        eq = jnp.where(sid == slotid, 1.0, 0.0) * keepf
        contrib = jnp.sum(eq * row, axis=0, keepdims=True)
        slots_r[...] = slots_r[...] + jnp.broadcast_to(contrib, (NE, S))
        cnt[0:1, 0:NE] = prev + jnp.sum(onehot, axis=0, keepdims=True)

    return pl.pallas_call(
        kfn, grid=grid,
        in_specs=[x_spec,
                  pl.BlockSpec((T, 128), lambda i: (i, 0)),
                  pl.BlockSpec((dlx, 128), lambda i: (0, 0)),
                  pl.BlockSpec((128, 128), lambda i: (0, 0))],
        out_specs=[pl.BlockSpec((T, NE), lambda i: (i, 0)),
                   pl.BlockSpec((T, NE), lambda i: (i, 0)),
                   pl.BlockSpec((NE, S), lambda i: (0, 0))],
        out_shape=[jax.ShapeDtypeStruct((Npad, NE), F32),
                   jax.ShapeDtypeStruct((Npad, NE), F32),
                   jax.ShapeDtypeStruct((NE, S), F32)],
        scratch_shapes=[pltpu.VMEM((8, 128), F32)],
    )(x3, posp, gx, gp)


def _eka_call(xd3, M):
    S = xd3.shape[0]
    STRIDE = S // NE

    def kfn(x_r, m_r, o_r):
        xb = x_r[...]
        mm = m_r[0]
        z = jnp.zeros((STRIDE, 1, 512), F32)
        pb = jnp.concatenate([z, xb, z], axis=1)
        for i in range(16):
            xc = jnp.concatenate(
                [pb[:, i, :], pb[:, i + 1, :], pb[:, i + 2, :]], axis=1)
            o_r[:, i, :] = _silu(_dot(xc, mm))

    return pl.pallas_call(
        kfn, grid=(NE,),
        in_specs=[pl.BlockSpec((STRIDE, 16, 512), lambda e: (e, 0, 0)),
                  pl.BlockSpec((1, 1536, 512), lambda e: (e, 0, 0))],
        out_specs=pl.BlockSpec((STRIDE, 16, 512), lambda e: (e, 0, 0)),
        out_shape=jax.ShapeDtypeStruct((S, 16, 512), F32),
    )(xd3, M)


def _ekb_call(xd, M):
    S, D = xd.shape
    STRIDE = S // NE

    def kfn(x_r, m_r, o_r):
        o_r[...] = _silu(_dot(x_r[...], m_r[0]))

    return pl.pallas_call(
        kfn, grid=(NE,),
        in_specs=[pl.BlockSpec((STRIDE, D), lambda e: (e, 0)),
                  pl.BlockSpec((1, D, D), lambda e: (e, 0, 0))],
        out_specs=pl.BlockSpec((STRIDE, D), lambda e: (e, 0)),
        out_shape=jax.ShapeDtypeStruct((S, D), F32),
    )(xd, M)


def _combine_call(tok, yg, coef, gam8):
    Npad, D = tok.shape
    T = 128

    def kfn(x_r, y_r, c_r, g_r, o_r):
        o_r[...] = x_r[...] + g_r[0:1, :] * c_r[:, 0:1] * y_r[...]

    return pl.pallas_call(
        kfn, grid=(Npad // T,),
        in_specs=[pl.BlockSpec((T, D), lambda i: (i, 0)),
                  pl.BlockSpec((T, D), lambda i: (i, 0)),
                  pl.BlockSpec((T, NE), lambda i: (i, 0)),
                  pl.BlockSpec((NE, D), lambda i: (0, 0))],
        out_specs=pl.BlockSpec((T, D), lambda i: (i, 0)),
        out_shape=jax.ShapeDtypeStruct((Npad, D), F32),
    )(tok, yg, coef, gam8)


def _pce_layer(tok, l, params):
    cfg = _PCE[l]
    C, ps, N, Npad = cfg["C"], cfg["ps"], cfg["N"], cfg["Npad"]
    ps2 = ps * ps
    D = ps2 * C
    S = NE * cfg["STRIDE"]
    gW = params["pce%d_gateW" % l]
    gb = params["pce%d_gateb" % l]
    if ps == 16:
        gx = jnp.tile(gW[:C] / 256.0, (16, 1))
    else:
        gx = jnp.tile(gW[:C] / float(ps2), (ps2, 1))
    gx = jnp.pad(gx, ((0, 0), (0, 120)))
    gp = jnp.zeros((128, 128), F32)
    gp = gp.at[0:FCH, 0:NE].set(gW[C:]).at[FCH, 0:NE].set(gb)
    posp = jnp.asarray(_POSP[cfg["hp"]])
    x3 = tok.reshape(Npad, 16, 512) if ps == 16 else tok
    meta, coef, slots = _router_call(x3, posp, gx, gp, cfg)
    slot_tok = slots[0].astype(jnp.int32)
    tidx = meta[:, 0].astype(jnp.int32)
    xd = _sc_gather(tok, slot_tok)
    W8 = params["pce%d_experts" % l]
    if ps == 16:
        y = _eka_call(xd.reshape(S, 16, 512), _build_M_row(W8)).reshape(S, D)
    elif ps == 1:
        y = _ekb_call(xd, jnp.transpose(W8[:, :, :, 1, 1], (0, 2, 1)))
    else:
        y = _ekb_call(xd, _build_M_std(W8, ps, C))
    yg = _sc_gather(y, tidx)
    gamD = jnp.tile(params["pce%d_gamma" % l], (ps2,))
    gam8 = jnp.broadcast_to(gamD[None, :], (NE, D))
    return _combine_call(tok, yg, coef, gam8)


def _extract(img, ps, cfg):
    B, H, W_, C = img.shape
    hp = H // ps
    t = img.reshape(B, hp, ps, hp, ps, C).transpose(0, 1, 3, 2, 4, 5)
    t = t.reshape(B * hp * hp, ps * ps * C)
    return jnp.pad(t, ((0, cfg["Npad"] - B * hp * hp), (0, 0)))


def _reassemble(tok, ps, cfg, C):
    B = BATCH
    hp = cfg["hp"]
    t = tok[:cfg["N"]].reshape(B, hp, hp, ps, ps, C)
    t = t.transpose(0, 1, 3, 2, 4, 5).reshape(B, hp * ps, hp * ps, C)
    return t


def _head_call(hx, params):
    g = jnp.tile(params["head_ln_g"][None, :], (8, 1))
    b = jnp.tile(params["head_ln_b"][None, :], (8, 1))
    hb = jnp.tile(params["head_b"][None, :], (8, 1))
    W = params["head_W"]

    def kfn(x_r, g_r, b_r, w_r, hb_r, o_r):
        f = jnp.mean(x_r[...], axis=1)
        mu = jnp.mean(f, axis=1, keepdims=True)
        var = jnp.mean((f - mu) ** 2, axis=1, keepdims=True)
        fn = (f - mu) * lax.rsqrt(var + 1e-5) * g_r[0:1, :] + b_r[0:1, :]
        o = _dot(fn, w_r[...]) + hb_r[0:1, :]
        o_r[...] = jnp.concatenate([o, jnp.zeros((4, 1000), F32)], axis=0)

    out = pl.pallas_call(
        kfn, grid=(1,),
        in_specs=[pl.BlockSpec((4, 784, 256), lambda i: (0, 0, 0)),
                  pl.BlockSpec((8, 256), lambda i: (0, 0)),
                  pl.BlockSpec((8, 256), lambda i: (0, 0)),
                  pl.BlockSpec((256, 1000), lambda i: (0, 0)),
                  pl.BlockSpec((8, 1000), lambda i: (0, 0))],
        out_specs=pl.BlockSpec((8, 1000), lambda i: (0, 0)),
        out_shape=jax.ShapeDtypeStruct((8, 1000), F32),
    )(hx, g, b, W, hb)
    return out[:4]


def kernel(X, params):
    x = jnp.transpose(X, (0, 2, 3, 1)).astype(F32)
    img = _stem_layer(x, params["stem_w"])
    ps = 16
    tok = None
    for l in range(8):
        if l in (2, 4, 6):
            img = _down_layer(img, params["down%d_w" % l],
                              params["down%d_skip" % l])
            ps = ps // 4 if l == 2 else max(1, ps // 2)
        else:
            cfg = _PCE[l]
            if l == 0 or (l - 1) in (2, 4, 6):
                tok = _extract(img, ps, cfg)
            tok = _pce_layer(tok, l, params)
            if (l + 1) in (2, 4, 6) or l == 7:
                img = _reassemble(tok, ps, cfg, cfg["C"])
    hx = img.reshape(BATCH, 784, 256)
    return _head_call(hx, params)


# speedup vs baseline: 1.7870x; 1.0392x over previous
import functools
import math

import numpy as np
import jax
import jax.numpy as jnp
from jax import lax
from jax.experimental import pallas as pl
from jax.experimental.pallas import tpu as pltpu
from jax.experimental.pallas import tpu_sc as plsc

F32 = jnp.float32
NE = 8
FCH = 18
BATCH = 4

_PCE = {
    0: dict(C=32, ps=16, hp=14, N=784, Npad=1024, Ccap=123, STRIDE=128),
    1: dict(C=32, ps=16, hp=14, N=784, Npad=1024, Ccap=123, STRIDE=128),
    3: dict(C=64, ps=4, hp=28, N=3136, Npad=3328, Ccap=490, STRIDE=512),
    5: dict(C=128, ps=2, hp=28, N=3136, Npad=3328, Ccap=490, STRIDE=512),
    7: dict(C=256, ps=1, hp=28, N=3136, Npad=3328, Ccap=490, STRIDE=512),
}


def _silu(x):
    return x * jax.nn.sigmoid(x)


def _dot(a, b):
    return lax.dot_general(a, b, (((a.ndim - 1,), (0,)), ((), ())),
                           preferred_element_type=F32)


def _mxdot(a, b):
    return lax.dot_general(a.astype(jnp.bfloat16), b.astype(jnp.bfloat16),
                           (((a.ndim - 1,), (0,)), ((), ())),
                           preferred_element_type=F32)


def _posfeat_np(hp):
    ys = (np.arange(hp) + 0.5) / hp
    yy, xx = np.meshgrid(ys, ys, indexing="ij")
    feats = [yy, xx]
    for f in range(4):
        wf = (2.0 ** f) * np.pi
        feats += [np.sin(wf * yy), np.cos(wf * yy),
                  np.sin(wf * xx), np.cos(wf * xx)]
    return np.stack(feats, 0).reshape(FCH, hp * hp).T.astype(np.float32)


_POSP = {}
for _hp, _N, _Npad in ((14, 784, 1024), (28, 3136, 3328)):
    _pf = np.tile(_posfeat_np(_hp), (BATCH, 1))
    _pp = np.zeros((_Npad, 128), np.float32)
    _pp[:_N, :FCH] = _pf
    _pp[:_N, FCH] = 1.0
    _POSP[_hp] = _pp


def _patch_adj_np(ps):
    A = np.zeros((9, ps * ps, ps * ps), np.float32)
    for pi in range(ps):
        for pj in range(ps):
            for di in (-1, 0, 1):
                for dj in (-1, 0, 1):
                    qi, qj = pi + di, pj + dj
                    if 0 <= qi < ps and 0 <= qj < ps:
                        k = (di + 1) * 3 + (dj + 1)
                        A[k, pi * ps + pj, qi * ps + qj] = 1.0
    return A


_ADJ = {ps: _patch_adj_np(ps) for ps in (2, 4)}
_AROW = np.zeros((3, 16, 16), np.float32)
for _pj in range(16):
    for _dj in (-1, 0, 1):
        _q = _pj + _dj
        if 0 <= _q < 16:
            _AROW[_dj + 1, _pj, _q] = 1.0


def _build_M_std(W8, ps, C):
    A = _ADJ[ps]
    Wk = jnp.transpose(W8, (0, 3, 4, 2, 1)).reshape(NE, 9, C, C)
    M = jnp.einsum("kpq,ekio->eqipo", jnp.asarray(A), Wk)
    return M.reshape(NE, ps * ps * C, ps * ps * C)


def _build_M_row(W8):
    Wk = jnp.transpose(W8, (0, 3, 4, 2, 1))
    M = jnp.einsum("jpq,edjio->edqipo", jnp.asarray(_AROW), Wk)
    return M.reshape(NE, 3 * 512, 512)


_NC, _NS = 2, 16
_NW = _NC * _NS


def _sc_gather(table, idx):
    V, D = table.shape
    B = idx.shape[0]
    assert B % _NW == 0 and D % 16 == 0
    rpw = B // _NW
    ch = min(rpw, 128)
    while ch > 0 and not (rpw % ch == 0 and ch * D * 4 <= 440_000):
        ch -= 8
    assert ch > 0
    nch = rpw // ch
    mesh = plsc.VectorSubcoreMesh(core_axis_name="c", subcore_axis_name="s")

    @functools.partial(
        pl.kernel, mesh=mesh,
        out_type=jax.ShapeDtypeStruct((B, D), table.dtype),
        scratch_types=[pltpu.VMEM((ch,), jnp.int32),
                       pltpu.VMEM((ch, D), table.dtype),
                       pltpu.SemaphoreType.DMA])
    def k(tab, ix, out, ixv, rows, sem):
        wid = lax.axis_index("s") * _NC + lax.axis_index("c")
        base = wid * rpw
        for c in range(nch):
            off = base + c * ch
            pltpu.sync_copy(ix.at[pl.ds(off, ch)], ixv)
            pltpu.async_copy(tab.at[ixv], rows, sem).wait()
            pltpu.sync_copy(rows, out.at[pl.ds(off, ch)])

    return k(table, idx)


def _stem_call(xp, w):
    B, RC = BATCH, 16
    nb = 224 // RC
    grid = (B, nb)
    sa = pl.BlockSpec((1, RC, 232, 3), lambda b, j: (b, j, 0, 0))
    sb = pl.BlockSpec((1, RC, 232, 3), lambda b, j: (b, j + 1, 0, 0))

    def kfn(a_r, b_r, w_r, o_r):
        rows = jnp.concatenate([a_r[0], b_r[0, 0:2]], axis=0)
        shifted = [rows[:, dj:dj + 224, :] for dj in range(3)]
        cols = []
        for di in range(3):
            for dj in range(3):
                cols.append(
                    shifted[dj][di:di + RC].reshape(RC * 224, 3))
        ic = jnp.concatenate(
            cols + [jnp.zeros((RC * 224, 5), F32)], axis=1)
        o_r[...] = _silu(_mxdot(ic, w_r[...]))[None]

    out = pl.pallas_call(
        kfn, grid=grid,
        in_specs=[sa, sb, pl.BlockSpec((32, 32), lambda b, j: (0, 0))],
        out_specs=pl.BlockSpec((1, RC * 224, 32), lambda b, j: (b, j, 0)),
        out_shape=jax.ShapeDtypeStruct((B, 224 * 224, 32), F32),
    )(xp, xp, w)
    return out.reshape(B, 224, 224, 32)


def _stem_layer(x, stem_w):
    xp = jnp.pad(x, ((0, 0), (1, 15), (1, 7), (0, 0)))
    w27 = jnp.transpose(stem_w, (2, 3, 1, 0)).reshape(27, 32)
    w27p = jnp.concatenate([w27, jnp.zeros((5, 32), F32)], axis=0)
    return _stem_call(xp, w27p)


def _down_call(xpv, wm, wsm, Hout, Wout, Cin, Co, RC):
    B = BATCH
    nb = Hout // RC
    grid = (B, nb)
    C2 = 2 * Cin
    sa = pl.BlockSpec((1, 2 * RC, Wout, C2), lambda b, j: (b, j, 0, 0))
    sb = pl.BlockSpec((1, 2 * RC, Wout, C2), lambda b, j: (b, j + 1, 0, 0))

    def kfn(a_r, b_r, wm_r, ws_r, o_r):
        rows = jnp.concatenate([a_r[0], b_r[0, 0:2]], axis=0)
        cols = []
        skip = None
        for di in range(3):
            rdi = rows[di:di + 2 * RC].reshape(RC, 2, Wout, C2)[:, 0]
            ev = rdi[:, :, 0:Cin]
            od = rdi[:, :, Cin:C2]
            om = jnp.concatenate(
                [jnp.zeros((RC, 1, Cin), F32), od[:, 0:Wout - 1, :]], axis=1)
            cols += [om.reshape(RC * Wout, Cin),
                     ev.reshape(RC * Wout, Cin),
                     od.reshape(RC * Wout, Cin)]
            if di == 1:
                skip = cols[4]
        ic = jnp.concatenate(cols, axis=1)
        y = _silu(_mxdot(ic, wm_r[...])) + _mxdot(skip, ws_r[...])
        o_r[...] = y[None]

    out = pl.pallas_call(
        kfn, grid=grid,
        in_specs=[sa, sb,
                  pl.BlockSpec((9 * Cin, Co), lambda b, j: (0, 0)),
                  pl.BlockSpec((Cin, Co), lambda b, j: (0, 0))],
        out_specs=pl.BlockSpec((1, RC * Wout, Co), lambda b, j: (b, j, 0)),
        out_shape=jax.ShapeDtypeStruct((B, Hout * Wout, Co), F32),
    )(xpv, xpv, wm, wsm)
    return out.reshape(B, Hout, Wout, Co)


def _down_layer(img, w, ws):
    B, H, W_, Cin = img.shape
    Hout, Wout = H // 2, W_ // 2
    Co = w.shape[0]
    RC = 28
    nb = Hout // RC
    hpad = (nb + 1) * 2 * RC
    xp = jnp.pad(img, ((0, 0), (1, hpad - H - 1), (0, 0), (0, 0)))
    xpv = xp.reshape(B, hpad, Wout, 2 * Cin)
    wm = jnp.transpose(w, (2, 3, 1, 0)).reshape(9 * Cin, Co)
    wsm = ws[:, :, 0, 0].T
    return _down_call(xpv, wm, wsm, Hout, Wout, Cin, Co, RC)


def _router_call(x3, posp, gx, gp, cfg):
    N, Npad, Ccap, STRIDE = cfg["N"], cfg["Npad"], cfg["Ccap"], cfg["STRIDE"]
    S = NE * STRIDE
    T = 128
    two = cfg["ps"] == 16
    grid = (Npad // T,)
    if two:
        x_spec = pl.BlockSpec((T, 16, 512), lambda i: (i, 0, 0))
        dlx = 512
    else:
        D = cfg["ps"] ** 2 * cfg["C"]
        x_spec = pl.BlockSpec((T, D), lambda i: (i, 0))
        dlx = D

    def kfn(x_r, pp_r, gx_r, gp_r, meta_r, coef_r, slots_r, cnt):
        i = pl.program_id(0)

        @pl.when(i == 0)
        def _init():
            cnt[...] = jnp.zeros_like(cnt)
            slots_r[...] = jnp.zeros_like(slots_r)

        xb = x_r[...]
        xs = jnp.sum(xb, axis=1) if two else xb
        logits = _dot(xs, gx_r[...]) + _dot(pp_r[...], gp_r[...])
        lg = logits[:, 0:NE]
        mx = jnp.max(lg, axis=1, keepdims=True)
        ex = jnp.exp(lg - mx)
        probs = ex / jnp.sum(ex, axis=1, keepdims=True)
        pmax = jnp.max(probs, axis=1, keepdims=True)
        lane = lax.broadcasted_iota(jnp.int32, (T, NE), 1).astype(F32)
        t1 = jnp.min(jnp.where(probs >= pmax, lane, 1e9), axis=1,
                     keepdims=True)
        row = (i * T + lax.broadcasted_iota(jnp.int32, (T, 1), 0)).astype(F32)
        valid = row < N
        onehot = jnp.where((lane == t1) & valid, 1.0, 0.0)
        ri = lax.broadcasted_iota(jnp.int32, (T, T), 0)
        ci_ = lax.broadcasted_iota(jnp.int32, (T, T), 1)
        tri = jnp.where(ri >= ci_, 1.0, 0.0)
        csum = _dot(tri, onehot)
        prev = cnt[0:1, 0:NE]
        ppos = jnp.sum((csum + prev) * onehot, axis=1, keepdims=True) - 1.0
        keep = (ppos < Ccap) & valid
        coefv = jnp.where(keep, pmax, 0.0)
        sp_ = jnp.clip(ppos, 0.0, Ccap - 1.0)
        slotid = jnp.where(valid, t1 * STRIDE + sp_, 0.0)
        meta_r[...] = jnp.broadcast_to(slotid, (T, NE)).astype(jnp.int32)
        coef_r[...] = jnp.broadcast_to(coefv, (T, NE))
        sid = lax.broadcasted_iota(jnp.int32, (T, S), 1).astype(F32)
        keepf = jnp.where(keep, 1.0, 0.0)
        eq = jnp.where(sid == slotid, 1.0, 0.0) * keepf
        contrib = jnp.sum(eq * row, axis=0, keepdims=True)
        slots_r[...] = slots_r[...] + jnp.broadcast_to(contrib, (NE, S))
        cnt[0:1, 0:NE] = prev + jnp.sum(onehot, axis=0, keepdims=True)

    return pl.pallas_call(
        kfn, grid=grid,
        in_specs=[x_spec,
                  pl.BlockSpec((T, 128), lambda i: (i, 0)),
                  pl.BlockSpec((dlx, 128), lambda i: (0, 0)),
                  pl.BlockSpec((128, 128), lambda i: (0, 0))],
        out_specs=[pl.BlockSpec((T, NE), lambda i: (i, 0)),
                   pl.BlockSpec((T, NE), lambda i: (i, 0)),
                   pl.BlockSpec((NE, S), lambda i: (0, 0))],
        out_shape=[jax.ShapeDtypeStruct((Npad, NE), jnp.int32),
                   jax.ShapeDtypeStruct((Npad, NE), F32),
                   jax.ShapeDtypeStruct((NE, S), F32)],
        scratch_shapes=[pltpu.VMEM((8, 128), F32)],
    )(x3, posp, gx, gp)


def _eka_call(xd3, M):
    S = xd3.shape[0]
    STRIDE = S // NE

    def kfn(x_r, m_r, o_r):
        xb = x_r[...]
        mm = m_r[0]
        z = jnp.zeros((STRIDE, 1, 512), F32)
        pb = jnp.concatenate([z, xb, z], axis=1)
        mmb = mm.astype(jnp.bfloat16)
        for i in range(16):
            xc = jnp.concatenate(
                [pb[:, i, :], pb[:, i + 1, :], pb[:, i + 2, :]], axis=1)
            y = lax.dot_general(xc.astype(jnp.bfloat16), mmb,
                                (((1,), (0,)), ((), ())),
                                preferred_element_type=F32)
            o_r[:, i, :] = _silu(y)

    return pl.pallas_call(
        kfn, grid=(NE,),
        in_specs=[pl.BlockSpec((STRIDE, 16, 512), lambda e: (e, 0, 0)),
                  pl.BlockSpec((1, 1536, 512), lambda e: (e, 0, 0))],
        out_specs=pl.BlockSpec((STRIDE, 16, 512), lambda e: (e, 0, 0)),
        out_shape=jax.ShapeDtypeStruct((S, 16, 512), F32),
    )(xd3, M)


def _ekb_call(xd, M):
    S, D = xd.shape
    STRIDE = S // NE

    def kfn(x_r, m_r, o_r):
        o_r[...] = _silu(_mxdot(x_r[...], m_r[0]))

    return pl.pallas_call(
        kfn, grid=(NE,),
        in_specs=[pl.BlockSpec((STRIDE, D), lambda e: (e, 0)),
                  pl.BlockSpec((1, D, D), lambda e: (e, 0, 0))],
        out_specs=pl.BlockSpec((STRIDE, D), lambda e: (e, 0)),
        out_shape=jax.ShapeDtypeStruct((S, D), F32),
    )(xd, M)


def _combine_call(tok, yg, coef, gam8):
    Npad, D = tok.shape
    T = 128

    def kfn(x_r, y_r, c_r, g_r, o_r):
        o_r[...] = x_r[...] + g_r[0:1, :] * c_r[:, 0:1] * y_r[...]

    return pl.pallas_call(
        kfn, grid=(Npad // T,),
        in_specs=[pl.BlockSpec((T, D), lambda i: (i, 0)),
                  pl.BlockSpec((T, D), lambda i: (i, 0)),
                  pl.BlockSpec((T, NE), lambda i: (i, 0)),
                  pl.BlockSpec((NE, D), lambda i: (0, 0))],
        out_specs=pl.BlockSpec((T, D), lambda i: (i, 0)),
        out_shape=jax.ShapeDtypeStruct((Npad, D), F32),
    )(tok, yg, coef, gam8)


def _pce_layer(tok, l, params):
    cfg = _PCE[l]
    C, ps, N, Npad = cfg["C"], cfg["ps"], cfg["N"], cfg["Npad"]
    ps2 = ps * ps
    D = ps2 * C
    S = NE * cfg["STRIDE"]
    gW = params["pce%d_gateW" % l]
    gb = params["pce%d_gateb" % l]
    if ps == 16:
        gx = jnp.tile(gW[:C] / 256.0, (16, 1))
    else:
        gx = jnp.tile(gW[:C] / float(ps2), (ps2, 1))
    gx = jnp.pad(gx, ((0, 0), (0, 120)))
    gp = jnp.zeros((128, 128), F32)
    gp = gp.at[0:FCH, 0:NE].set(gW[C:]).at[FCH, 0:NE].set(gb)
    posp = jnp.asarray(_POSP[cfg["hp"]])
    x3 = tok.reshape(Npad, 16, 512) if ps == 16 else tok
    meta, coef, slots = _router_call(x3, posp, gx, gp, cfg)
    slot_tok = slots[0].astype(jnp.int32)
    tidx = meta[:, 0]
    xd = _sc_gather(tok, slot_tok)
    W8 = params["pce%d_experts" % l]
    if ps == 16:
        y = _eka_call(xd.reshape(S, 16, 512), _build_M_row(W8)).reshape(S, D)
    elif ps == 1:
        y = _ekb_call(xd, jnp.transpose(W8[:, :, :, 1, 1], (0, 2, 1)))
    else:
        y = _ekb_call(xd, _build_M_std(W8, ps, C))
    yg = _sc_gather(y, tidx)
    gamD = jnp.tile(params["pce%d_gamma" % l], (ps2,))
    gam8 = jnp.broadcast_to(gamD[None, :], (NE, D))
    return _combine_call(tok, yg, coef, gam8)


def _extract(img, ps, cfg):
    B, H, W_, C = img.shape
    hp = H // ps
    t = img.reshape(B, hp, ps, hp, ps, C).transpose(0, 1, 3, 2, 4, 5)
    t = t.reshape(B * hp * hp, ps * ps * C)
    return jnp.pad(t, ((0, cfg["Npad"] - B * hp * hp), (0, 0)))


def _reassemble(tok, ps, cfg, C):
    B = BATCH
    hp = cfg["hp"]
    t = tok[:cfg["N"]].reshape(B, hp, hp, ps, ps, C)
    t = t.transpose(0, 1, 3, 2, 4, 5).reshape(B, hp * ps, hp * ps, C)
    return t


def _head_call(hx, params):
    g = jnp.tile(params["head_ln_g"][None, :], (8, 1))
    b = jnp.tile(params["head_ln_b"][None, :], (8, 1))
    hb = jnp.tile(params["head_b"][None, :], (8, 1))
    W = params["head_W"]

    def kfn(x_r, g_r, b_r, w_r, hb_r, o_r):
        f = jnp.mean(x_r[...], axis=1)
        mu = jnp.mean(f, axis=1, keepdims=True)
        var = jnp.mean((f - mu) ** 2, axis=1, keepdims=True)
        fn = (f - mu) * lax.rsqrt(var + 1e-5) * g_r[0:1, :] + b_r[0:1, :]
        o = _dot(fn, w_r[...]) + hb_r[0:1, :]
        o_r[...] = jnp.concatenate([o, jnp.zeros((4, 1000), F32)], axis=0)

    out = pl.pallas_call(
        kfn, grid=(1,),
        in_specs=[pl.BlockSpec((4, 784, 256), lambda i: (0, 0, 0)),
                  pl.BlockSpec((8, 256), lambda i: (0, 0)),
                  pl.BlockSpec((8, 256), lambda i: (0, 0)),
                  pl.BlockSpec((256, 1000), lambda i: (0, 0)),
                  pl.BlockSpec((8, 1000), lambda i: (0, 0))],
        out_specs=pl.BlockSpec((8, 1000), lambda i: (0, 0)),
        out_shape=jax.ShapeDtypeStruct((8, 1000), F32),
    )(hx, g, b, W, hb)
    return out[:4]


def kernel(X, params):
    x = jnp.transpose(X, (0, 2, 3, 1)).astype(F32)
    img = _stem_layer(x, params["stem_w"])
    ps = 16
    tok = None
    for l in range(8):
        if l in (2, 4, 6):
            img = _down_layer(img, params["down%d_w" % l],
                              params["down%d_skip" % l])
            ps = ps // 4 if l == 2 else max(1, ps // 2)
        else:
            cfg = _PCE[l]
            if l == 0 or (l - 1) in (2, 4, 6):
                tok = _extract(img, ps, cfg)
            tok = _pce_layer(tok, l, params)
            if (l + 1) in (2, 4, 6) or l == 7:
                img = _reassemble(tok, ps, cfg, cfg["C"])
    hx = img.reshape(BATCH, 784, 256)
    return _head_call(hx, params)
